# baseline (speedup 1.0000x reference)
import functools

import jax
import jax.numpy as jnp
from jax import lax
from jax.experimental import pallas as pl
from jax.experimental.pallas import tpu as pltpu
from jax.experimental.pallas import tpu_sc as plsc

B = 16384
ED = 128


_NC = 2
_NS = 16
_NW = _NC * _NS
_BPW = B // _NW
_IDXW = 128
_NCHUNK = _BPW // _IDXW


def _gather_body(ut, it, uid, iid, ue, ie, idx_v, rows_v, sem):
    wid = lax.axis_index("s") * _NC + lax.axis_index("c")
    base = wid * _BPW
    r0 = wid * _NCHUNK
    pltpu.sync_copy(uid.at[pl.ds(r0, _NCHUNK)], idx_v)
    for j in range(_NCHUNK):
        pltpu.async_copy(ut.at[idx_v.at[j]],
                         rows_v.at[pl.ds(j * _IDXW, _IDXW)], sem).wait()
    pltpu.sync_copy(rows_v, ue.at[pl.ds(base, _BPW)])
    pltpu.sync_copy(iid.at[pl.ds(r0, _NCHUNK)], idx_v)
    for j in range(_NCHUNK):
        pltpu.async_copy(it.at[idx_v.at[j]],
                         rows_v.at[pl.ds(j * _IDXW, _IDXW)], sem).wait()
    pltpu.sync_copy(rows_v, ie.at[pl.ds(base, _BPW)])


@functools.cache
def _make_sc_gather():
    return pl.kernel(
        _gather_body,
        out_type=(jax.ShapeDtypeStruct((B, ED), jnp.float32),
                  jax.ShapeDtypeStruct((B, ED), jnp.float32)),
        mesh=plsc.VectorSubcoreMesh(core_axis_name="c", subcore_axis_name="s"),
        scratch_types=[
            pltpu.VMEM((_NCHUNK, _IDXW), jnp.int32),
            pltpu.VMEM((_BPW, ED), jnp.float32),
            pltpu.SemaphoreType.DMA,
        ],
    )


_TB = 512


def _mlp_body(ue, ie, ts, dow, w0u, w0i, wts, day8, w0d,
              b0, g0, be0, m0, v0,
              w1, b1, g1, be1, m1, v1,
              w2, b2, g2, be2, m2, v2,
              wft, bf, out):
    f32 = jnp.float32
    bf16 = jnp.bfloat16
    x_u = ue[...].astype(bf16)
    x_i = ie[...].astype(bf16)
    h = jnp.dot(x_u, w0u[...].astype(bf16), preferred_element_type=f32)
    h += jnp.dot(x_i, w0i[...].astype(bf16), preferred_element_type=f32)
    h += ts[...] * wts[...]
    day_w = jnp.dot(day8[...], w0d[...], preferred_element_type=f32)
    oh = (dow[...] == lax.broadcasted_iota(jnp.int32, (1, 8), 1)).astype(f32)
    h += jnp.dot(oh, day_w, preferred_element_type=f32)
    h = (h + b0[...] - m0[...]) * (g0[...] * lax.rsqrt(v0[...] + 1e-5)) + be0[...]
    h = jnp.maximum(h, 0.0)

    h = jnp.dot(h.astype(bf16), w1[...].astype(bf16), preferred_element_type=f32)
    h = (h + b1[...] - m1[...]) * (g1[...] * lax.rsqrt(v1[...] + 1e-5)) + be1[...]
    h = jnp.maximum(h, 0.0)

    h = jnp.dot(h.astype(bf16), w2[...].astype(bf16), preferred_element_type=f32)
    h = (h + b2[...] - m2[...]) * (g2[...] * lax.rsqrt(v2[...] + 1e-5)) + be2[...]
    h = jnp.maximum(h, 0.0)

    z = jnp.sum(h * wft[...], axis=1, keepdims=True) + bf[...]
    out[...] = 5.0 / (1.0 + jnp.exp(-z))


def _full(shape):
    return pl.BlockSpec(shape, lambda i: (0, 0))


_mlp = pl.pallas_call(
    _mlp_body,
    grid=(B // _TB,),
    in_specs=[
        pl.BlockSpec((_TB, ED), lambda i: (i, 0)),
        pl.BlockSpec((_TB, ED), lambda i: (i, 0)),
        pl.BlockSpec((_TB, 1), lambda i: (i, 0)),
        pl.BlockSpec((_TB, 1), lambda i: (i, 0)),
        _full((ED, 1024)),
        _full((ED, 1024)),
        _full((1, 1024)),
        _full((8, ED)),
        _full((ED, 1024)),
        _full((1, 1024)), _full((1, 1024)), _full((1, 1024)), _full((1, 1024)), _full((1, 1024)),
        _full((1024, 512)),
        _full((1, 512)), _full((1, 512)), _full((1, 512)), _full((1, 512)), _full((1, 512)),
        _full((512, 256)),
        _full((1, 256)), _full((1, 256)), _full((1, 256)), _full((1, 256)), _full((1, 256)),
        _full((1, 256)),
        _full((1, 1)),
    ],
    out_specs=pl.BlockSpec((_TB, 1), lambda i: (i, 0)),
    out_shape=jax.ShapeDtypeStruct((B, 1), jnp.float32),
    compiler_params=pltpu.CompilerParams(
        dimension_semantics=("parallel",),
    ),
)


def kernel(user_ids, item_ids, timestamps, day_of_week,
           user_table, item_table, day_table,
           W0, b0, g0, be0, m0, v0,
           W1, b1, g1, be1, m1, v1,
           W2, b2, g2, be2, m2, v2,
           Wf, bf):
    uid2 = user_ids.astype(jnp.int32).reshape(B // _IDXW, _IDXW)
    iid2 = item_ids.astype(jnp.int32).reshape(B // _IDXW, _IDXW)
    ue, ie = _make_sc_gather()(user_table, item_table, uid2, iid2)

    w0u = W0[:ED]
    w0i = W0[ED:2 * ED]
    wts = W0[2 * ED:2 * ED + 1]
    w0d = jnp.zeros((ED, 1024), jnp.float32).at[:4].set(W0[2 * ED + 1:])
    day8 = jnp.zeros((8, ED), jnp.float32).at[:7, :4].set(day_table)

    out = _mlp(
        ue, ie, timestamps.reshape(B, 1), day_of_week.astype(jnp.int32).reshape(B, 1),
        w0u, w0i, wts, day8, w0d,
        b0.reshape(1, -1), g0.reshape(1, -1), be0.reshape(1, -1), m0.reshape(1, -1), v0.reshape(1, -1),
        W1,
        b1.reshape(1, -1), g1.reshape(1, -1), be1.reshape(1, -1), m1.reshape(1, -1), v1.reshape(1, -1),
        W2,
        b2.reshape(1, -1), g2.reshape(1, -1), be2.reshape(1, -1), m2.reshape(1, -1), v2.reshape(1, -1),
        Wf.reshape(1, -1), bf.reshape(1, 1),
    )
    return out


# speedup vs baseline: 1.1136x; 1.1136x over previous
import functools

import jax
import jax.numpy as jnp
from jax import lax
from jax.experimental import pallas as pl
from jax.experimental.pallas import tpu as pltpu
from jax.experimental.pallas import tpu_sc as plsc

B = 16384
ED = 128


_NC = 2
_NS = 16
_NW = _NC * _NS
_BPW = B // _NW
_IDXW = 128
_NCHUNK = _BPW // _IDXW


def _gather_body(ut, it, uid, iid, ue, ie, idx_v, rows_v, sem):
    wid = lax.axis_index("s") * _NC + lax.axis_index("c")
    base = wid * _BPW
    r0 = wid * _NCHUNK
    pltpu.sync_copy(uid.at[pl.ds(r0, _NCHUNK)], idx_v)
    for j in range(_NCHUNK):
        pltpu.async_copy(ut.at[idx_v.at[j]],
                         rows_v.at[pl.ds(j * _IDXW, _IDXW)], sem).wait()
    pltpu.sync_copy(rows_v, ue.at[pl.ds(base, _BPW)])
    pltpu.sync_copy(iid.at[pl.ds(r0, _NCHUNK)], idx_v)
    for j in range(_NCHUNK):
        pltpu.async_copy(it.at[idx_v.at[j]],
                         rows_v.at[pl.ds(j * _IDXW, _IDXW)], sem).wait()
    pltpu.sync_copy(rows_v, ie.at[pl.ds(base, _BPW)])


@functools.cache
def _make_sc_gather():
    return pl.kernel(
        _gather_body,
        out_type=(jax.ShapeDtypeStruct((B, ED), jnp.float32),
                  jax.ShapeDtypeStruct((B, ED), jnp.float32)),
        mesh=plsc.VectorSubcoreMesh(core_axis_name="c", subcore_axis_name="s"),
        scratch_types=[
            pltpu.VMEM((_NCHUNK, _IDXW), jnp.int32),
            pltpu.VMEM((_BPW, ED), jnp.float32),
            pltpu.SemaphoreType.DMA,
        ],
    )


_TB = 1024


def _mlp_body(ue, ie, ts, dow, w01, w0ext, sel16,
              b0, g0, be0, m0, v0,
              w1, b1, g1, be1, m1, v1,
              w2, b2, g2, be2, m2, v2,
              wft, bf, out):
    f32 = jnp.float32
    iota16 = lax.broadcasted_iota(jnp.int32, (1, 16), 1)
    oh = (dow[...] == iota16).astype(f32)
    e = jnp.where(iota16 == 8, ts[...], oh)
    ew = jnp.dot(sel16[...], w0ext[...], preferred_element_type=f32)

    xcat = jnp.concatenate([ue[...], ie[...]], axis=1)
    h = jnp.dot(xcat, w01[...], preferred_element_type=f32)
    h += jnp.dot(e, ew, preferred_element_type=f32)
    s = g0[...] * lax.rsqrt(v0[...] + 1e-5)
    t = (b0[...] - m0[...]) * s + be0[...]
    h = jnp.maximum(h * s + t, 0.0)

    h = jnp.dot(h, w1[...], preferred_element_type=f32)
    s = g1[...] * lax.rsqrt(v1[...] + 1e-5)
    t = (b1[...] - m1[...]) * s + be1[...]
    h = jnp.maximum(h * s + t, 0.0)

    h = jnp.dot(h, w2[...], preferred_element_type=f32)
    s = g2[...] * lax.rsqrt(v2[...] + 1e-5)
    t = (b2[...] - m2[...]) * s + be2[...]
    h = jnp.maximum(h * s + t, 0.0)

    z = jnp.sum(h * wft[...], axis=1, keepdims=True) + bf[...]
    r = 5.0 / (1.0 + jnp.exp(-z))
    out[...] = jnp.reshape(r, (_TB // 128, 128))


def _full(shape):
    return pl.BlockSpec(shape, lambda i: (0, 0))


_mlp = pl.pallas_call(
    _mlp_body,
    grid=(B // _TB,),
    in_specs=[
        pl.BlockSpec((_TB, ED), lambda i: (i, 0)),
        pl.BlockSpec((_TB, ED), lambda i: (i, 0)),
        pl.BlockSpec((_TB, 1), lambda i: (i, 0)),
        pl.BlockSpec((_TB, 1), lambda i: (i, 0)),
        _full((2 * ED, 1024)),
        _full((8, 1024)),
        _full((16, 8)),
        _full((1, 1024)), _full((1, 1024)), _full((1, 1024)), _full((1, 1024)), _full((1, 1024)),
        _full((1024, 512)),
        _full((1, 512)), _full((1, 512)), _full((1, 512)), _full((1, 512)), _full((1, 512)),
        _full((512, 256)),
        _full((1, 256)), _full((1, 256)), _full((1, 256)), _full((1, 256)), _full((1, 256)),
        _full((1, 256)),
        _full((1, 1)),
    ],
    out_specs=pl.BlockSpec((_TB // 128, 128), lambda i: (i, 0)),
    out_shape=jax.ShapeDtypeStruct((B // 128, 128), jnp.float32),
    compiler_params=pltpu.CompilerParams(
        dimension_semantics=("parallel",),
    ),
)


def kernel(user_ids, item_ids, timestamps, day_of_week,
           user_table, item_table, day_table,
           W0, b0, g0, be0, m0, v0,
           W1, b1, g1, be1, m1, v1,
           W2, b2, g2, be2, m2, v2,
           Wf, bf):
    uid2 = user_ids.astype(jnp.int32).reshape(B // _IDXW, _IDXW)
    iid2 = item_ids.astype(jnp.int32).reshape(B // _IDXW, _IDXW)
    ue, ie = _make_sc_gather()(user_table, item_table, uid2, iid2)

    w01 = W0[:2 * ED]
    w0ext = jnp.pad(W0[2 * ED:], ((0, 3), (0, 0)))
    sel16 = (jnp.zeros((16, 8), jnp.float32)
             .at[:7, 1:5].set(day_table).at[8, 0].set(1.0))

    out = _mlp(
        ue, ie, timestamps.reshape(B, 1), day_of_week.astype(jnp.int32).reshape(B, 1),
        w01, w0ext, sel16,
        b0.reshape(1, -1), g0.reshape(1, -1), be0.reshape(1, -1), m0.reshape(1, -1), v0.reshape(1, -1),
        W1,
        b1.reshape(1, -1), g1.reshape(1, -1), be1.reshape(1, -1), m1.reshape(1, -1), v1.reshape(1, -1),
        W2,
        b2.reshape(1, -1), g2.reshape(1, -1), be2.reshape(1, -1), m2.reshape(1, -1), v2.reshape(1, -1),
        Wf.reshape(1, -1), bf.reshape(1, 1),
    )
    return out.reshape(B, 1)


# speedup vs baseline: 1.1357x; 1.0198x over previous
import functools

import jax
import jax.numpy as jnp
from jax import lax
from jax.experimental import pallas as pl
from jax.experimental.pallas import tpu as pltpu
from jax.experimental.pallas import tpu_sc as plsc

B = 16384
ED = 128
_C = 2
_BC = B // _C


_NC = 2
_NS = 16
_NW = _NC * _NS
_BPW = _BC // _NW
_IDXW = 128
_NCHUNK = _BPW // _IDXW


def _gather_body(ut, it, uid, iid, ue, ie, idx_v, rows_v, sem):
    wid = lax.axis_index("s") * _NC + lax.axis_index("c")
    base = wid * _BPW
    r0 = wid * _NCHUNK
    pltpu.sync_copy(uid.at[pl.ds(r0, _NCHUNK)], idx_v)
    for j in range(_NCHUNK):
        pltpu.async_copy(ut.at[idx_v.at[j]],
                         rows_v.at[pl.ds(j * _IDXW, _IDXW)], sem).wait()
    pltpu.sync_copy(rows_v, ue.at[pl.ds(base, _BPW)])
    pltpu.sync_copy(iid.at[pl.ds(r0, _NCHUNK)], idx_v)
    for j in range(_NCHUNK):
        pltpu.async_copy(it.at[idx_v.at[j]],
                         rows_v.at[pl.ds(j * _IDXW, _IDXW)], sem).wait()
    pltpu.sync_copy(rows_v, ie.at[pl.ds(base, _BPW)])


@functools.cache
def _make_sc_gather():
    return pl.kernel(
        _gather_body,
        out_type=(jax.ShapeDtypeStruct((_BC, ED), jnp.float32),
                  jax.ShapeDtypeStruct((_BC, ED), jnp.float32)),
        mesh=plsc.VectorSubcoreMesh(core_axis_name="c", subcore_axis_name="s"),
        scratch_types=[
            pltpu.VMEM((_NCHUNK, _IDXW), jnp.int32),
            pltpu.VMEM((_BPW, ED), jnp.float32),
            pltpu.SemaphoreType.DMA,
        ],
    )


_TB = 1024


def _mlp_body(ue, ie, e, w01, w0ext, sel16,
              b0, g0, be0, m0, v0,
              w1, b1, g1, be1, m1, v1,
              w2, b2, g2, be2, m2, v2,
              wft, bf, out):
    f32 = jnp.float32
    ew = jnp.dot(sel16[...], w0ext[...], preferred_element_type=f32)

    xcat = jnp.concatenate([ue[...], ie[...]], axis=1)
    h = jnp.dot(xcat, w01[...], preferred_element_type=f32)
    h += jnp.dot(e[...], ew, preferred_element_type=f32)
    s = g0[...] * lax.rsqrt(v0[...] + 1e-5)
    t = (b0[...] - m0[...]) * s + be0[...]
    h = jnp.maximum(h * s + t, 0.0)

    h = jnp.dot(h, w1[...], preferred_element_type=f32)
    s = g1[...] * lax.rsqrt(v1[...] + 1e-5)
    t = (b1[...] - m1[...]) * s + be1[...]
    h = jnp.maximum(h * s + t, 0.0)

    h = jnp.dot(h, w2[...], preferred_element_type=f32)
    s = g2[...] * lax.rsqrt(v2[...] + 1e-5)
    t = (b2[...] - m2[...]) * s + be2[...]
    h = jnp.maximum(h * s + t, 0.0)

    z = jnp.sum(h * wft[...], axis=1, keepdims=True) + bf[...]
    r = 5.0 / (1.0 + jnp.exp(-z))
    out[...] = jnp.reshape(r, (_TB // 128, 128))


def _full(shape):
    return pl.BlockSpec(shape, lambda i: (0, 0))


_mlp = pl.pallas_call(
    _mlp_body,
    grid=(_BC // _TB,),
    in_specs=[
        pl.BlockSpec((_TB, ED), lambda i: (i, 0)),
        pl.BlockSpec((_TB, ED), lambda i: (i, 0)),
        pl.BlockSpec((_TB, 16), lambda i: (i, 0)),
        _full((2 * ED, 1024)),
        _full((8, 1024)),
        _full((16, 8)),
        _full((1, 1024)), _full((1, 1024)), _full((1, 1024)), _full((1, 1024)), _full((1, 1024)),
        _full((1024, 512)),
        _full((1, 512)), _full((1, 512)), _full((1, 512)), _full((1, 512)), _full((1, 512)),
        _full((512, 256)),
        _full((1, 256)), _full((1, 256)), _full((1, 256)), _full((1, 256)), _full((1, 256)),
        _full((1, 256)),
        _full((1, 1)),
    ],
    out_specs=pl.BlockSpec((_TB // 128, 128), lambda i: (i, 0)),
    out_shape=jax.ShapeDtypeStruct((_BC // 128, 128), jnp.float32),
    compiler_params=pltpu.CompilerParams(
        dimension_semantics=("parallel",),
    ),
)


def kernel(user_ids, item_ids, timestamps, day_of_week,
           user_table, item_table, day_table,
           W0, b0, g0, be0, m0, v0,
           W1, b1, g1, be1, m1, v1,
           W2, b2, g2, be2, m2, v2,
           Wf, bf):
    uid2 = user_ids.astype(jnp.int32).reshape(B // _IDXW, _IDXW)
    iid2 = item_ids.astype(jnp.int32).reshape(B // _IDXW, _IDXW)

    cols = jnp.arange(16)[None, :]
    e = jnp.where(cols == 8, timestamps[:, None],
                  (day_of_week[:, None] == cols).astype(jnp.float32))

    w01 = W0[:2 * ED]
    w0ext = jnp.pad(W0[2 * ED:], ((0, 3), (0, 0)))
    sel16 = (jnp.zeros((16, 8), jnp.float32)
             .at[:7, 1:5].set(day_table).at[8, 0].set(1.0))

    bn = (b0.reshape(1, -1), g0.reshape(1, -1), be0.reshape(1, -1), m0.reshape(1, -1), v0.reshape(1, -1),
          W1,
          b1.reshape(1, -1), g1.reshape(1, -1), be1.reshape(1, -1), m1.reshape(1, -1), v1.reshape(1, -1),
          W2,
          b2.reshape(1, -1), g2.reshape(1, -1), be2.reshape(1, -1), m2.reshape(1, -1), v2.reshape(1, -1),
          Wf.reshape(1, -1), bf.reshape(1, 1))

    gather = _make_sc_gather()
    rows_per_chunk = _BC // _IDXW
    outs = []
    for c in range(_C):
        ue, ie = gather(user_table, item_table,
                        uid2[c * rows_per_chunk:(c + 1) * rows_per_chunk],
                        iid2[c * rows_per_chunk:(c + 1) * rows_per_chunk])
        e_c = e[c * _BC:(c + 1) * _BC]
        outs.append(_mlp(ue, ie, e_c, w01, w0ext, sel16, *bn))
    return jnp.concatenate(outs, axis=0).reshape(B, 1)


# speedup vs baseline: 1.2807x; 1.1277x over previous
import functools

import jax
import jax.numpy as jnp
from jax import lax
from jax.experimental import pallas as pl
from jax.experimental.pallas import tpu as pltpu
from jax.experimental.pallas import tpu_sc as plsc

B = 16384
ED = 128
_C = 2
_BC = B // _C


_NC = 2
_NS = 16
_NW = _NC * _NS
_BPW = _BC // _NW
_IDXW = 128
_NCHUNK = _BPW // _IDXW


def _gather_body(ut, it, uid, iid, ue, ie, idx_v, rows_v, sem):
    wid = lax.axis_index("s") * _NC + lax.axis_index("c")
    base = wid * _BPW
    r0 = wid * _NCHUNK
    pltpu.sync_copy(uid.at[pl.ds(r0, _NCHUNK)], idx_v)
    for j in range(_NCHUNK):
        pltpu.async_copy(ut.at[idx_v.at[j]],
                         rows_v.at[pl.ds(j * _IDXW, _IDXW)], sem).wait()
    pltpu.sync_copy(rows_v, ue.at[pl.ds(base, _BPW)])
    pltpu.sync_copy(iid.at[pl.ds(r0, _NCHUNK)], idx_v)
    for j in range(_NCHUNK):
        pltpu.async_copy(it.at[idx_v.at[j]],
                         rows_v.at[pl.ds(j * _IDXW, _IDXW)], sem).wait()
    pltpu.sync_copy(rows_v, ie.at[pl.ds(base, _BPW)])


@functools.cache
def _make_sc_gather():
    return pl.kernel(
        _gather_body,
        out_type=(jax.ShapeDtypeStruct((_BC, ED), jnp.float32),
                  jax.ShapeDtypeStruct((_BC, ED), jnp.float32)),
        mesh=plsc.VectorSubcoreMesh(core_axis_name="c", subcore_axis_name="s"),
        scratch_types=[
            pltpu.VMEM((_NCHUNK, _IDXW), jnp.int32),
            pltpu.VMEM((_BPW, ED), jnp.float32),
            pltpu.SemaphoreType.DMA,
        ],
    )


_TB = 1024


def _mlp_body(ue, ie, e, w01, w0ext, sel16,
              b0, g0, be0, m0, v0,
              w1, b1, g1, be1, m1, v1,
              w2, b2, g2, be2, m2, v2,
              wf, bf, out):
    f32 = jnp.float32
    ew = jnp.dot(sel16[...], w0ext[...], preferred_element_type=f32)

    xcat = jnp.concatenate([ue[...], ie[...]], axis=1)
    h = jnp.dot(xcat, w01[...], preferred_element_type=f32)
    h += jnp.dot(e[...], ew, preferred_element_type=f32)
    s = g0[...] * lax.rsqrt(v0[...] + 1e-5)
    t = (b0[...] - m0[...]) * s + be0[...]
    h = jnp.maximum(h * s + t, 0.0)

    h = jnp.dot(h, w1[...], preferred_element_type=f32)
    s = g1[...] * lax.rsqrt(v1[...] + 1e-5)
    t = (b1[...] - m1[...]) * s + be1[...]
    h = jnp.maximum(h * s + t, 0.0)

    h = jnp.dot(h, w2[...], preferred_element_type=f32)
    s = g2[...] * lax.rsqrt(v2[...] + 1e-5)
    t = (b2[...] - m2[...]) * s + be2[...]
    h = jnp.maximum(h * s + t, 0.0)

    z = jnp.dot(h, wf[...], preferred_element_type=f32)
    z8 = jnp.reshape(z, (_TB // 128, 128)) + bf[...]
    out[...] = 5.0 / (1.0 + jnp.exp(-z8))


def _full(shape):
    return pl.BlockSpec(shape, lambda i: (0, 0))


_mlp = pl.pallas_call(
    _mlp_body,
    grid=(_BC // _TB,),
    in_specs=[
        pl.BlockSpec((_TB, ED), lambda i: (i, 0)),
        pl.BlockSpec((_TB, ED), lambda i: (i, 0)),
        pl.BlockSpec((_TB, 16), lambda i: (i, 0)),
        _full((2 * ED, 1024)),
        _full((8, 1024)),
        _full((16, 8)),
        _full((1, 1024)), _full((1, 1024)), _full((1, 1024)), _full((1, 1024)), _full((1, 1024)),
        _full((1024, 512)),
        _full((1, 512)), _full((1, 512)), _full((1, 512)), _full((1, 512)), _full((1, 512)),
        _full((512, 256)),
        _full((1, 256)), _full((1, 256)), _full((1, 256)), _full((1, 256)), _full((1, 256)),
        _full((2 * ED, 1)),
        _full((1, 1)),
    ],
    out_specs=pl.BlockSpec((_TB // 128, 128), lambda i: (i, 0)),
    out_shape=jax.ShapeDtypeStruct((_BC // 128, 128), jnp.float32),
    compiler_params=pltpu.CompilerParams(
        dimension_semantics=("parallel",),
    ),
)


def kernel(user_ids, item_ids, timestamps, day_of_week,
           user_table, item_table, day_table,
           W0, b0, g0, be0, m0, v0,
           W1, b1, g1, be1, m1, v1,
           W2, b2, g2, be2, m2, v2,
           Wf, bf):
    uid2 = user_ids.astype(jnp.int32).reshape(B // _IDXW, _IDXW)
    iid2 = item_ids.astype(jnp.int32).reshape(B // _IDXW, _IDXW)

    cols = jnp.arange(16)[None, :]
    e = jnp.where(cols == 8, timestamps[:, None],
                  (day_of_week[:, None] == cols).astype(jnp.float32))

    w01 = W0[:2 * ED]
    w0ext = jnp.pad(W0[2 * ED:], ((0, 3), (0, 0)))
    sel16 = (jnp.zeros((16, 8), jnp.float32)
             .at[:7, 1:5].set(day_table).at[8, 0].set(1.0))

    bn = (b0.reshape(1, -1), g0.reshape(1, -1), be0.reshape(1, -1), m0.reshape(1, -1), v0.reshape(1, -1),
          W1,
          b1.reshape(1, -1), g1.reshape(1, -1), be1.reshape(1, -1), m1.reshape(1, -1), v1.reshape(1, -1),
          W2,
          b2.reshape(1, -1), g2.reshape(1, -1), be2.reshape(1, -1), m2.reshape(1, -1), v2.reshape(1, -1),
          Wf, bf.reshape(1, 1))

    gather = _make_sc_gather()
    rows_per_chunk = _BC // _IDXW
    outs = []
    for c in range(_C):
        ue, ie = gather(user_table, item_table,
                        uid2[c * rows_per_chunk:(c + 1) * rows_per_chunk],
                        iid2[c * rows_per_chunk:(c + 1) * rows_per_chunk])
        e_c = e[c * _BC:(c + 1) * _BC]
        outs.append(_mlp(ue, ie, e_c, w01, w0ext, sel16, *bn))
    return jnp.concatenate(outs, axis=0).reshape(B, 1)


# speedup vs baseline: 1.3106x; 1.0233x over previous
import functools

import jax
import jax.numpy as jnp
from jax import lax
from jax.experimental import pallas as pl
from jax.experimental.pallas import tpu as pltpu
from jax.experimental.pallas import tpu_sc as plsc

B = 16384
ED = 128
_C = 2
_BC = B // _C


_NC = 2
_NS = 16
_NW = _NC * _NS
_BPW = _BC // _NW
_IDXW = 128
_NCHUNK = _BPW // _IDXW


def _gather_body(ut, it, uid, iid, ue, ie, idx_v, rows_v, sem):
    wid = lax.axis_index("s") * _NC + lax.axis_index("c")
    base = wid * _BPW
    r0 = wid * _NCHUNK
    pltpu.sync_copy(uid.at[pl.ds(r0, _NCHUNK)], idx_v)
    for j in range(_NCHUNK):
        pltpu.async_copy(ut.at[idx_v.at[j]],
                         rows_v.at[pl.ds(j * _IDXW, _IDXW)], sem).wait()
    pltpu.sync_copy(rows_v, ue.at[pl.ds(base, _BPW)])
    pltpu.sync_copy(iid.at[pl.ds(r0, _NCHUNK)], idx_v)
    for j in range(_NCHUNK):
        pltpu.async_copy(it.at[idx_v.at[j]],
                         rows_v.at[pl.ds(j * _IDXW, _IDXW)], sem).wait()
    pltpu.sync_copy(rows_v, ie.at[pl.ds(base, _BPW)])


@functools.cache
def _make_sc_gather():
    return pl.kernel(
        _gather_body,
        out_type=(jax.ShapeDtypeStruct((_BC, ED), jnp.float32),
                  jax.ShapeDtypeStruct((_BC, ED), jnp.float32)),
        mesh=plsc.VectorSubcoreMesh(core_axis_name="c", subcore_axis_name="s"),
        scratch_types=[
            pltpu.VMEM((_NCHUNK, _IDXW), jnp.int32),
            pltpu.VMEM((_BPW, ED), jnp.float32),
            pltpu.SemaphoreType.DMA,
        ],
    )


_TB = 2048


def _mlp_body(ue, ie, e, w01, w0ext, sel16,
              b0, g0, be0, m0, v0,
              w1, b1, g1, be1, m1, v1,
              w2, b2, g2, be2, m2, v2,
              wf, bf, out):
    f32 = jnp.float32
    ew = jnp.dot(sel16[...], w0ext[...], preferred_element_type=f32)

    xcat = jnp.concatenate([ue[...], ie[...]], axis=1)
    h = jnp.dot(xcat, w01[...], preferred_element_type=f32)
    h += jnp.dot(e[...], ew, preferred_element_type=f32)
    s = g0[...] * lax.rsqrt(v0[...] + 1e-5)
    t = (b0[...] - m0[...]) * s + be0[...]
    h = jnp.maximum(h * s + t, 0.0)

    h = jnp.dot(h, w1[...], preferred_element_type=f32)
    s = g1[...] * lax.rsqrt(v1[...] + 1e-5)
    t = (b1[...] - m1[...]) * s + be1[...]
    h = jnp.maximum(h * s + t, 0.0)

    h = jnp.dot(h, w2[...], preferred_element_type=f32)
    s = g2[...] * lax.rsqrt(v2[...] + 1e-5)
    t = (b2[...] - m2[...]) * s + be2[...]
    h = jnp.maximum(h * s + t, 0.0)

    z = jnp.dot(h, wf[...], preferred_element_type=f32)
    z8 = jnp.reshape(z, (_TB // 128, 128)) + bf[...]
    out[...] = 5.0 / (1.0 + jnp.exp(-z8))


def _full(shape):
    return pl.BlockSpec(shape, lambda i: (0, 0))


_mlp = pl.pallas_call(
    _mlp_body,
    grid=(_BC // _TB,),
    in_specs=[
        pl.BlockSpec((_TB, ED), lambda i: (i, 0)),
        pl.BlockSpec((_TB, ED), lambda i: (i, 0)),
        pl.BlockSpec((_TB, 16), lambda i: (i, 0)),
        _full((2 * ED, 1024)),
        _full((8, 1024)),
        _full((16, 8)),
        _full((1, 1024)), _full((1, 1024)), _full((1, 1024)), _full((1, 1024)), _full((1, 1024)),
        _full((1024, 512)),
        _full((1, 512)), _full((1, 512)), _full((1, 512)), _full((1, 512)), _full((1, 512)),
        _full((512, 256)),
        _full((1, 256)), _full((1, 256)), _full((1, 256)), _full((1, 256)), _full((1, 256)),
        _full((2 * ED, 1)),
        _full((1, 1)),
    ],
    out_specs=pl.BlockSpec((_TB // 128, 128), lambda i: (i, 0)),
    out_shape=jax.ShapeDtypeStruct((_BC // 128, 128), jnp.float32),
    compiler_params=pltpu.CompilerParams(
        dimension_semantics=("parallel",),
    ),
)


def kernel(user_ids, item_ids, timestamps, day_of_week,
           user_table, item_table, day_table,
           W0, b0, g0, be0, m0, v0,
           W1, b1, g1, be1, m1, v1,
           W2, b2, g2, be2, m2, v2,
           Wf, bf):
    uid2 = user_ids.astype(jnp.int32).reshape(B // _IDXW, _IDXW)
    iid2 = item_ids.astype(jnp.int32).reshape(B // _IDXW, _IDXW)

    cols = jnp.arange(16)[None, :]
    e = jnp.where(cols == 8, timestamps[:, None],
                  (day_of_week[:, None] == cols).astype(jnp.float32))

    w01 = W0[:2 * ED]
    w0ext = jnp.pad(W0[2 * ED:], ((0, 3), (0, 0)))
    sel16 = (jnp.zeros((16, 8), jnp.float32)
             .at[:7, 1:5].set(day_table).at[8, 0].set(1.0))

    bn = (b0.reshape(1, -1), g0.reshape(1, -1), be0.reshape(1, -1), m0.reshape(1, -1), v0.reshape(1, -1),
          W1,
          b1.reshape(1, -1), g1.reshape(1, -1), be1.reshape(1, -1), m1.reshape(1, -1), v1.reshape(1, -1),
          W2,
          b2.reshape(1, -1), g2.reshape(1, -1), be2.reshape(1, -1), m2.reshape(1, -1), v2.reshape(1, -1),
          Wf, bf.reshape(1, 1))

    gather = _make_sc_gather()
    rows_per_chunk = _BC // _IDXW
    outs = []
    for c in range(_C):
        ue, ie = gather(user_table, item_table,
                        uid2[c * rows_per_chunk:(c + 1) * rows_per_chunk],
                        iid2[c * rows_per_chunk:(c + 1) * rows_per_chunk])
        e_c = e[c * _BC:(c + 1) * _BC]
        outs.append(_mlp(ue, ie, e_c, w01, w0ext, sel16, *bn))
    return jnp.concatenate(outs, axis=0).reshape(B, 1)


# speedup vs baseline: 1.3162x; 1.0042x over previous
import functools

import jax
import jax.numpy as jnp
from jax import lax
from jax.experimental import pallas as pl
from jax.experimental.pallas import tpu as pltpu
from jax.experimental.pallas import tpu_sc as plsc

B = 16384
ED = 128
_C = 2
_BC = B // _C


_NC = 2
_NS = 16
_NW = _NC * _NS
_BPW = _BC // _NW
_IDXW = 128
_NCHUNK = _BPW // _IDXW


def _gather_body(ut, it, uid, iid, xc, idx_v, rows_v, sem):
    wid = lax.axis_index("s") * _NC + lax.axis_index("c")
    base = wid * _BPW
    r0 = wid * _NCHUNK
    pltpu.sync_copy(uid.at[pl.ds(r0, _NCHUNK)], idx_v)
    for j in range(_NCHUNK):
        pltpu.async_copy(ut.at[idx_v.at[j]],
                         rows_v.at[pl.ds(j * _IDXW, _IDXW)], sem).wait()
    pltpu.sync_copy(rows_v, xc.at[pl.ds(base, _BPW), pl.ds(0, ED)])
    pltpu.sync_copy(iid.at[pl.ds(r0, _NCHUNK)], idx_v)
    for j in range(_NCHUNK):
        pltpu.async_copy(it.at[idx_v.at[j]],
                         rows_v.at[pl.ds(j * _IDXW, _IDXW)], sem).wait()
    pltpu.sync_copy(rows_v, xc.at[pl.ds(base, _BPW), pl.ds(ED, ED)])


@functools.cache
def _make_sc_gather():
    return pl.kernel(
        _gather_body,
        out_type=jax.ShapeDtypeStruct((_BC, 2 * ED), jnp.float32),
        mesh=plsc.VectorSubcoreMesh(core_axis_name="c", subcore_axis_name="s"),
        scratch_types=[
            pltpu.VMEM((_NCHUNK, _IDXW), jnp.int32),
            pltpu.VMEM((_BPW, ED), jnp.float32),
            pltpu.SemaphoreType.DMA,
        ],
    )


_TB = 2048


def _mlp_body(xc, e, w01, w0ext, sel16,
              b0, g0, be0, m0, v0,
              w1, b1, g1, be1, m1, v1,
              w2, b2, g2, be2, m2, v2,
              wf, bf, out):
    f32 = jnp.float32
    ew = jnp.dot(sel16[...], w0ext[...], preferred_element_type=f32)

    h = jnp.dot(xc[...], w01[...], preferred_element_type=f32)
    h += jnp.dot(e[...], ew, preferred_element_type=f32)
    s = g0[...] * lax.rsqrt(v0[...] + 1e-5)
    t = (b0[...] - m0[...]) * s + be0[...]
    h = jnp.maximum(h * s + t, 0.0)

    h = jnp.dot(h, w1[...], preferred_element_type=f32)
    s = g1[...] * lax.rsqrt(v1[...] + 1e-5)
    t = (b1[...] - m1[...]) * s + be1[...]
    h = jnp.maximum(h * s + t, 0.0)

    h = jnp.dot(h, w2[...], preferred_element_type=f32)
    s = g2[...] * lax.rsqrt(v2[...] + 1e-5)
    t = (b2[...] - m2[...]) * s + be2[...]
    h = jnp.maximum(h * s + t, 0.0)

    z = jnp.dot(h, wf[...], preferred_element_type=f32)
    z8 = jnp.reshape(z, (_TB // 128, 128)) + bf[...]
    out[...] = 5.0 / (1.0 + jnp.exp(-z8))


def _full(shape):
    return pl.BlockSpec(shape, lambda i: (0, 0))


_mlp = pl.pallas_call(
    _mlp_body,
    grid=(_BC // _TB,),
    in_specs=[
        pl.BlockSpec((_TB, 2 * ED), lambda i: (i, 0)),
        pl.BlockSpec((_TB, 16), lambda i: (i, 0)),
        _full((2 * ED, 1024)),
        _full((8, 1024)),
        _full((16, 8)),
        _full((1, 1024)), _full((1, 1024)), _full((1, 1024)), _full((1, 1024)), _full((1, 1024)),
        _full((1024, 512)),
        _full((1, 512)), _full((1, 512)), _full((1, 512)), _full((1, 512)), _full((1, 512)),
        _full((512, 256)),
        _full((1, 256)), _full((1, 256)), _full((1, 256)), _full((1, 256)), _full((1, 256)),
        _full((2 * ED, 1)),
        _full((1, 1)),
    ],
    out_specs=pl.BlockSpec((_TB // 128, 128), lambda i: (i, 0)),
    out_shape=jax.ShapeDtypeStruct((_BC // 128, 128), jnp.float32),
    compiler_params=pltpu.CompilerParams(
        dimension_semantics=("parallel",),
    ),
)


def kernel(user_ids, item_ids, timestamps, day_of_week,
           user_table, item_table, day_table,
           W0, b0, g0, be0, m0, v0,
           W1, b1, g1, be1, m1, v1,
           W2, b2, g2, be2, m2, v2,
           Wf, bf):
    uid2 = user_ids.astype(jnp.int32).reshape(B // _IDXW, _IDXW)
    iid2 = item_ids.astype(jnp.int32).reshape(B // _IDXW, _IDXW)

    cols = jnp.arange(16)[None, :]
    e = jnp.where(cols == 8, timestamps[:, None],
                  (day_of_week[:, None] == cols).astype(jnp.float32))

    w01 = W0[:2 * ED]
    w0ext = jnp.pad(W0[2 * ED:], ((0, 3), (0, 0)))
    sel16 = (jnp.zeros((16, 8), jnp.float32)
             .at[:7, 1:5].set(day_table).at[8, 0].set(1.0))

    bn = (b0.reshape(1, -1), g0.reshape(1, -1), be0.reshape(1, -1), m0.reshape(1, -1), v0.reshape(1, -1),
          W1,
          b1.reshape(1, -1), g1.reshape(1, -1), be1.reshape(1, -1), m1.reshape(1, -1), v1.reshape(1, -1),
          W2,
          b2.reshape(1, -1), g2.reshape(1, -1), be2.reshape(1, -1), m2.reshape(1, -1), v2.reshape(1, -1),
          Wf, bf.reshape(1, 1))

    gather = _make_sc_gather()
    rows_per_chunk = _BC // _IDXW
    outs = []
    for c in range(_C):
        xc = gather(user_table, item_table,
                    uid2[c * rows_per_chunk:(c + 1) * rows_per_chunk],
                    iid2[c * rows_per_chunk:(c + 1) * rows_per_chunk])
        e_c = e[c * _BC:(c + 1) * _BC]
        outs.append(_mlp(xc, e_c, w01, w0ext, sel16, *bn))
    return jnp.concatenate(outs, axis=0).reshape(B, 1)
